# baseline (speedup 1.0000x reference)
import jax
import jax.numpy as jnp
from jax import lax
from jax.experimental import pallas as pl
from jax.experimental.pallas import tpu as pltpu
from jax.experimental.pallas import tpu_sc as plsc

N_NODES = 10000
N_EDGES = 320000
D = 128

NC = 2
NS = 16
NW = NC * NS

CHUNK = 64
NCHUNK = 157
E_PAD = NW * NCHUNK * CHUNK
NPAD = 10112
ROWS_PER_TILE = NPAD // NS


def _scatter_gather_kernel(x_hbm, src_hbm, dst_hbm, zero_hbm, h2_hbm,
                           src_v, dst_v, rows_v, acc, sem):
    c = lax.axis_index("c")
    s = lax.axis_index("s")
    wid = s * NC + c

    pltpu.sync_copy(src_hbm.at[wid], src_v)
    pltpu.sync_copy(dst_hbm.at[wid], dst_v)

    r0 = s * ROWS_PER_TILE
    pltpu.sync_copy(zero_hbm.at[pl.ds(r0, ROWS_PER_TILE)],
                    acc.at[pl.ds(r0, ROWS_PER_TILE)])
    plsc.subcore_barrier()

    def body(j, carry):
        pltpu.async_copy(x_hbm.at[src_v.at[j]], rows_v, sem).wait()
        pltpu.sync_copy(rows_v, acc.at[dst_v.at[j]], add=True)
        return carry

    lax.fori_loop(0, NCHUNK, body, 0)

    plsc.subcore_barrier()
    pltpu.sync_copy(acc.at[pl.ds(r0, ROWS_PER_TILE)],
                    h2_hbm.at[c, pl.ds(r0, ROWS_PER_TILE)])


@jax.jit
def _segment_sum_sc(x, src, dst, zero):
    mesh = plsc.VectorSubcoreMesh(core_axis_name="c", subcore_axis_name="s")
    return pl.kernel(
        _scatter_gather_kernel,
        out_type=jax.ShapeDtypeStruct((NC, NPAD, D), jnp.float32),
        mesh=mesh,
        scratch_types=[
            pltpu.VMEM((NCHUNK, CHUNK), jnp.int32),
            pltpu.VMEM((NCHUNK, CHUNK), jnp.int32),
            pltpu.VMEM((CHUNK, D), jnp.float32),
            pltpu.VMEM_SHARED((NPAD, D), jnp.float32),
            pltpu.SemaphoreType.DMA,
        ],
    )(x, src, dst, zero)


def _linear_body(h2_ref, w_ref, b_ref, o_ref):
    h = h2_ref[0] + h2_ref[1]
    o_ref[...] = lax.dot_general(
        h, w_ref[...], (((1,), (1,)), ((), ())),
        preferred_element_type=jnp.float32) + b_ref[...]


@jax.jit
def _linear_tc(h2, W, b2):
    blk = 1000
    grid = N_NODES // blk
    return pl.pallas_call(
        _linear_body,
        grid=(grid,),
        in_specs=[
            pl.BlockSpec((NC, blk, D), lambda i: (0, i, 0)),
            pl.BlockSpec((D, D), lambda i: (0, 0)),
            pl.BlockSpec((1, D), lambda i: (0, 0)),
        ],
        out_specs=pl.BlockSpec((blk, D), lambda i: (i, 0)),
        out_shape=jax.ShapeDtypeStruct((N_NODES, D), jnp.float32),
    )(h2, W, b2)


def kernel(inputs, edge_index, W, b):
    n_pad = E_PAD - N_EDGES
    src = jnp.concatenate(
        [edge_index[0], jnp.zeros((n_pad,), jnp.int32)]
    ).reshape(NW, NCHUNK, CHUNK)
    pad_dst = N_NODES + jnp.arange(n_pad, dtype=jnp.int32) % (NPAD - N_NODES)
    dst = jnp.concatenate(
        [edge_index[1], pad_dst]
    ).reshape(NW, NCHUNK, CHUNK)
    zero = jnp.zeros((NPAD, D), jnp.float32)
    h2 = _segment_sum_sc(inputs, src, dst, zero)
    return _linear_tc(h2, W, b.reshape(1, D))


# speedup vs baseline: 2.1110x; 2.1110x over previous
import jax
import jax.numpy as jnp
from jax import lax
from jax.experimental import pallas as pl
from jax.experimental.pallas import tpu as pltpu
from jax.experimental.pallas import tpu_sc as plsc

N_NODES = 10000
N_EDGES = 320000
D = 128

NC = 2
NS = 16
NW = NC * NS

CHUNK = 80
NCHUNK = 125
E_PAD = NW * NCHUNK * CHUNK
NPAD = 10112
ROWS_PER_TILE = NPAD // NS


def _scatter_gather_kernel(x_hbm, pk_hbm, zero_hbm, h2_hbm,
                           pk_v, sidx0, sidx1, didx0, didx1,
                           rows0, rows1, acc, sem0, sem1):
    c = lax.axis_index("c")
    s = lax.axis_index("s")
    wid = s * NC + c

    sidx = (sidx0, sidx1)
    didx = (didx0, didx1)
    bufs = (rows0, rows1)
    sems = (sem0, sem1)

    pltpu.sync_copy(pk_hbm.at[wid], pk_v)

    r0 = s * ROWS_PER_TILE
    pltpu.sync_copy(zero_hbm.at[pl.ds(r0, ROWS_PER_TILE)],
                    acc.at[pl.ds(r0, ROWS_PER_TILE)])
    plsc.subcore_barrier()

    def unpack(j, b):
        for t in range(CHUNK // 16):
            v = pk_v[j, pl.ds(t * 16, 16)]
            sidx[b][pl.ds(t * 16, 16)] = jnp.bitwise_and(v, 0xFFFF)
            didx[b][pl.ds(t * 16, 16)] = lax.shift_right_logical(v, 16)

    def gather(b):
        return pltpu.async_copy(x_hbm.at[sidx[b]], bufs[b], sems[b])

    def gather_wait(b):
        pltpu.make_async_copy(x_hbm.at[sidx[b]], bufs[b], sems[b]).wait()

    unpack(0, 0)
    unpack(1, 1)
    gather(0)
    gather(1)

    def body(i2, carry):
        for b in range(2):
            j = i2 * 2 + b
            gather_wait(b)
            pltpu.sync_copy(bufs[b], acc.at[didx[b]], add=True)
            jn = lax.rem(j + 2, NCHUNK)
            unpack(jn, b)
            gather(b)
        return carry

    lax.fori_loop(0, NCHUNK // 2, body, 0)

    gather_wait(0)
    pltpu.sync_copy(bufs[0], acc.at[didx[0]], add=True)
    gather_wait(1)

    plsc.subcore_barrier()
    pltpu.sync_copy(acc.at[pl.ds(r0, ROWS_PER_TILE)],
                    h2_hbm.at[c, pl.ds(r0, ROWS_PER_TILE)])


@jax.jit
def _segment_sum_sc(x, pk, zero):
    mesh = plsc.VectorSubcoreMesh(core_axis_name="c", subcore_axis_name="s")
    return pl.kernel(
        _scatter_gather_kernel,
        out_type=jax.ShapeDtypeStruct((NC, NPAD, D), jnp.float32),
        mesh=mesh,
        scratch_types=[
            pltpu.VMEM((NCHUNK, CHUNK), jnp.int32),
            pltpu.VMEM((CHUNK,), jnp.int32),
            pltpu.VMEM((CHUNK,), jnp.int32),
            pltpu.VMEM((CHUNK,), jnp.int32),
            pltpu.VMEM((CHUNK,), jnp.int32),
            pltpu.VMEM((CHUNK, D), jnp.float32),
            pltpu.VMEM((CHUNK, D), jnp.float32),
            pltpu.VMEM_SHARED((NPAD, D), jnp.float32),
            pltpu.SemaphoreType.DMA,
            pltpu.SemaphoreType.DMA,
        ],
    )(x, pk, zero)


def _linear_body(h2_ref, w_ref, b_ref, o_ref):
    h = h2_ref[0] + h2_ref[1]
    o_ref[...] = lax.dot_general(
        h, w_ref[...], (((1,), (1,)), ((), ())),
        preferred_element_type=jnp.float32) + b_ref[...]


@jax.jit
def _linear_tc(h2, W, b2):
    blk = 1000
    grid = N_NODES // blk
    return pl.pallas_call(
        _linear_body,
        grid=(grid,),
        in_specs=[
            pl.BlockSpec((NC, blk, D), lambda i: (0, i, 0)),
            pl.BlockSpec((D, D), lambda i: (0, 0)),
            pl.BlockSpec((1, D), lambda i: (0, 0)),
        ],
        out_specs=pl.BlockSpec((blk, D), lambda i: (i, 0)),
        out_shape=jax.ShapeDtypeStruct((N_NODES, D), jnp.float32),
    )(h2, W, b2)


def kernel(inputs, edge_index, W, b):
    n_pad = E_PAD - N_EDGES
    src = jnp.concatenate(
        [edge_index[0], jnp.zeros((n_pad,), jnp.int32)])
    pad_dst = N_NODES + jnp.arange(n_pad, dtype=jnp.int32) % (NPAD - N_NODES)
    dst = jnp.concatenate([edge_index[1], pad_dst])
    pk = (src | (dst << 16)).reshape(NW, NCHUNK, CHUNK)
    zero = jnp.zeros((NPAD, D), jnp.float32)
    h2 = _segment_sum_sc(inputs, pk, zero)
    return _linear_tc(h2, W, b.reshape(1, D))
